# baseline (speedup 1.0000x reference)
import jax
import jax.numpy as jnp
from jax import lax
from jax.experimental import pallas as pl
from jax.experimental.pallas import tpu as pltpu, tpu_sc as plsc

_INFO = plsc.get_sparse_core_info()
_NC, _NS = _INFO.num_cores, _INFO.num_subcores
_NW = _NC * _NS

_B = 4 * 8192
_V = 256
_D = 1024
_DSPLIT = 4
_DT = _D // _DSPLIT
_IG = _NW // _DSPLIT

_BA = _B // 2
_BB = _B - _BA
_CA = 16
_PER_WA = _BA // _NW
_STEPS_A = _PER_WA // _CA
_PER_WB = _BB // _IG
_CB = 64
_STEPS_B = _PER_WB // _CB


def _body(idx_hbm, table_a, table_b, out_hbm,
          idx_a, idx_b, table_v, rows0, rows1, ga0, ga1, semb):
    rows = (rows0, rows1)
    gsems = (ga0, ga1)
    wid = lax.axis_index("s") * _NC + lax.axis_index("c")
    ig = wid // _DSPLIT
    ds = wid % _DSPLIT
    abase = wid * _PER_WA
    bbase = _BA + ig * _PER_WB

    pltpu.make_async_copy(table_b.at[ds], table_v, semb).start()
    pltpu.make_async_copy(idx_hbm.at[pl.ds(abase, _PER_WA)], idx_a, ga0).start()
    pltpu.make_async_copy(idx_hbm.at[pl.ds(bbase, _PER_WB)], idx_b, ga1).start()
    pltpu.make_async_copy(table_b.at[ds], table_v, semb).wait()
    pltpu.make_async_copy(idx_hbm.at[pl.ds(abase, _PER_WA)], idx_a, ga0).wait()
    pltpu.make_async_copy(idx_hbm.at[pl.ds(bbase, _PER_WB)], idx_b, ga1).wait()

    def gather(g, b):
        return pltpu.make_async_copy(
            table_a.at[idx_a.at[pl.ds(g * _CA, _CA)]], rows[b], gsems[b])

    def emit(g, b):
        pltpu.sync_copy(rows[b], out_hbm.at[pl.ds(abase + g * _CA, _CA)])

    for b in range(2):
        gather(b, b).start()

    def fireb(g):
        for j16 in range(_CB // 16):
            ids16 = idx_b[pl.ds(g * _CB + j16 * 16, 16)]
            for j in range(16):
                p = bbase + g * _CB + j16 * 16 + j
                pltpu.make_async_copy(
                    table_v.at[ids16[j]], out_hbm.at[p, ds], semb).start()

    def drainb():
        pltpu.make_async_copy(
            out_hbm.at[pl.ds(bbase, _CA)], rows0, semb).wait()

    fireb(0)

    def bchunk(g, carry):
        fireb(g)
        drainb()
        return carry

    lax.fori_loop(1, _STEPS_B, bchunk, 0)
    drainb()

    def achunk(jj, carry):
        for b in range(2):
            g = jj * 2 + b
            gather(g, b).wait()
            emit(g, b)
            gather(g + 2, b).start()
        return carry

    lax.fori_loop(0, _STEPS_A // 2 - 1, achunk, 0)
    for b in range(2):
        g = _STEPS_A - 2 + b
        gather(g, b).wait()
        emit(g, b)


@jax.jit
def _embed_lookup(ids_flat, table_a, table_b):
    mesh = plsc.VectorSubcoreMesh(core_axis_name="c", subcore_axis_name="s")
    run = pl.kernel(
        _body,
        out_type=jax.ShapeDtypeStruct((_B, _DSPLIT, _DT), jnp.float32),
        mesh=mesh,
        compiler_params=pltpu.CompilerParams(
            use_tc_tiling_on_sc=False, needs_layout_passes=False),
        scratch_types=[
            pltpu.VMEM((_PER_WA,), jnp.int32),
            pltpu.VMEM((_PER_WB,), jnp.int32),
            pltpu.VMEM((_V, _DT), jnp.float32),
            pltpu.VMEM((_CA, _DSPLIT, _DT), jnp.float32),
            pltpu.VMEM((_CA, _DSPLIT, _DT), jnp.float32),
            pltpu.SemaphoreType.DMA,
            pltpu.SemaphoreType.DMA,
            pltpu.SemaphoreType.DMA,
        ],
    )
    return run(ids_flat, table_a, table_b)


def kernel(input_ids, attention_mask, embed):
    ids_flat = input_ids.reshape(-1).astype(jnp.int32)
    table_a = embed.reshape(_V, _DSPLIT, _DT)
    table_b = table_a.transpose(1, 0, 2)
    out = _embed_lookup(ids_flat, table_a, table_b)
    return out.reshape(input_ids.shape[0], input_ids.shape[1], _D)


# speedup vs baseline: 1.2098x; 1.2098x over previous
import jax
import jax.numpy as jnp
from jax import lax
from jax.experimental import pallas as pl
from jax.experimental.pallas import tpu as pltpu, tpu_sc as plsc

_INFO = plsc.get_sparse_core_info()
_NC, _NS = _INFO.num_cores, _INFO.num_subcores
_NW = _NC * _NS

_B = 4 * 8192
_V = 256
_D = 1024
_DSPLIT = 4
_DT = _D // _DSPLIT
_IG = _NW // _DSPLIT
_PER_G = _B // _IG
_C = 16
_STEPS = _PER_G // _C


def _body(idx_hbm, table_hbm, out_hbm, idx_v, table_v, drain_v, isem, tsem, sem):
    wid = lax.axis_index("s") * _NC + lax.axis_index("c")
    ig = wid // _DSPLIT
    ds = wid % _DSPLIT
    gbase = ig * _PER_G
    dcol = ds * _DT

    pltpu.make_async_copy(table_hbm.at[ds], table_v, tsem).start()
    pltpu.make_async_copy(idx_hbm.at[pl.ds(gbase, _PER_G)], idx_v, isem).start()
    pltpu.make_async_copy(table_hbm.at[ds], table_v, tsem).wait()
    pltpu.make_async_copy(idx_hbm.at[pl.ds(gbase, _PER_G)], idx_v, isem).wait()

    def fire(g):
        ids16 = idx_v[pl.ds(g * _C, _C)]
        for j in range(_C):
            pltpu.make_async_copy(
                table_v.at[ids16[j]],
                out_hbm.at[gbase + g * _C + j, pl.ds(dcol, _DT)], sem).start()

    def drain():
        pltpu.make_async_copy(
            out_hbm.at[pl.ds(gbase, _C), pl.ds(dcol, _DT)], drain_v, sem).wait()

    fire(0)

    def chunk(g, carry):
        fire(g)
        drain()
        return carry

    lax.fori_loop(1, _STEPS, chunk, 0)
    drain()


@jax.jit
def _embed_lookup(ids_flat, table_t):
    mesh = plsc.VectorSubcoreMesh(core_axis_name="c", subcore_axis_name="s")
    run = pl.kernel(
        _body,
        out_type=jax.ShapeDtypeStruct((_B, _D), jnp.float32),
        mesh=mesh,
        compiler_params=pltpu.CompilerParams(
            use_tc_tiling_on_sc=False, needs_layout_passes=False),
        scratch_types=[
            pltpu.VMEM((_PER_G,), jnp.int32),
            pltpu.VMEM((_V, _DT), jnp.float32),
            pltpu.VMEM((_C, _DT), jnp.float32),
            pltpu.SemaphoreType.DMA,
            pltpu.SemaphoreType.DMA,
            pltpu.SemaphoreType.DMA,
        ],
    )
    return run(ids_flat, table_t)


def kernel(input_ids, attention_mask, embed):
    ids_flat = input_ids.reshape(-1).astype(jnp.int32)
    table_t = embed.reshape(_V, _DSPLIT, _DT).transpose(1, 0, 2)
    out = _embed_lookup(ids_flat, table_t)
    return out.reshape(input_ids.shape[0], input_ids.shape[1], _D)


# speedup vs baseline: 3.6632x; 3.0280x over previous
import jax
import jax.numpy as jnp
from jax import lax
from jax.experimental import pallas as pl
from jax.experimental.pallas import tpu as pltpu, tpu_sc as plsc

_INFO = plsc.get_sparse_core_info()
_NC, _NS = _INFO.num_cores, _INFO.num_subcores
_NW = _NC * _NS

_B = 4 * 8192
_V = 256
_D = 1024
_DSPLIT = 4
_DT = _D // _DSPLIT
_IG = _NW // _DSPLIT
_PER_G = _B // _IG
_C = 16
_STEPS = _PER_G // _C


def _body(idx_hbm, table_hbm, out_hbm, idx_v, table_v, drain_v, isem, tsem, sem):
    wid = lax.axis_index("s") * _NC + lax.axis_index("c")
    ig = wid // _DSPLIT
    ds = wid % _DSPLIT
    gbase = ig * _PER_G
    dcol = ds * _DT

    pltpu.make_async_copy(table_hbm.at[ds], table_v, tsem).start()
    pltpu.make_async_copy(idx_hbm.at[pl.ds(gbase, _PER_G)], idx_v, isem).start()
    pltpu.make_async_copy(table_hbm.at[ds], table_v, tsem).wait()
    pltpu.make_async_copy(idx_hbm.at[pl.ds(gbase, _PER_G)], idx_v, isem).wait()

    def fire(g):
        ids16 = idx_v[pl.ds(g * _C, _C)]
        for j in range(_C):
            pltpu.make_async_copy(
                table_v.at[ids16[j]],
                out_hbm.at[gbase + g * _C + j, pl.ds(dcol, _DT)], sem).start()

    def drain():
        pltpu.make_async_copy(
            out_hbm.at[pl.ds(gbase, _C), pl.ds(dcol, _DT)], drain_v, sem).wait()

    fire(0)

    def chunk(g, carry):
        fire(g)
        drain()
        return carry

    lax.fori_loop(1, _STEPS, chunk, 0)
    drain()


@jax.jit
def _embed_lookup(ids_flat, table_t):
    mesh = plsc.VectorSubcoreMesh(core_axis_name="c", subcore_axis_name="s")
    run = pl.kernel(
        _body,
        out_type=jax.ShapeDtypeStruct((_B, _D), jnp.float32),
        mesh=mesh,
        scratch_types=[
            pltpu.VMEM((_PER_G,), jnp.int32),
            pltpu.VMEM((_V, _DT), jnp.float32),
            pltpu.VMEM((_C, _DT), jnp.float32),
            pltpu.SemaphoreType.DMA,
            pltpu.SemaphoreType.DMA,
            pltpu.SemaphoreType.DMA,
        ],
    )
    return run(ids_flat, table_t)


def kernel(input_ids, attention_mask, embed):
    ids_flat = input_ids.reshape(-1).astype(jnp.int32)
    table_t = embed.reshape(_V, _DSPLIT, _DT).transpose(1, 0, 2)
    out = _embed_lookup(ids_flat, table_t)
    return out.reshape(input_ids.shape[0], input_ids.shape[1], _D)
